# baseline (speedup 1.0000x reference)
import functools

import jax
import jax.numpy as jnp
from jax import lax
from jax.experimental import pallas as pl
from jax.experimental.pallas import tpu as pltpu
from jax.experimental.pallas import tpu_sc as plsc

N = 50000
NPAD = 50048
E = 800000
R = 1000
D = 64
B = 4096
W80 = 80

NC = 2
NS = 16
NW = NC * NS
NCH = 4
WEA = 25024
CA = WEA // NCH
WEB = E - 31 * WEA
CB = WEB // NCH
Q = 128
BUFSZ = 2 * CA + Q + 32
AUGROWS = 2008
TRASH_AUG = 2000
ACCROWS = 4352
TRASH_SLOT = B
PACK_TRASH = TRASH_SLOT * 2048 + TRASH_AUG

_mesh = plsc.VectorSubcoreMesh(core_axis_name="c", subcore_axis_name="s")
_sc_params = pltpu.CompilerParams(needs_layout_passes=False,
                                  use_tc_tiling_on_sc=False)


def _popcount(mask):
    cnt = plsc.all_reduce_population_count(mask)
    return cnt[0] if getattr(cnt, "ndim", 0) else cnt


@functools.partial(
    pl.kernel,
    out_type=jax.ShapeDtypeStruct((NC, B, W80), jnp.float32),
    mesh=_mesh,
    compiler_params=_sc_params,
    scratch_types=[
        pltpu.VMEM((NPAD,), jnp.int32),
        pltpu.VMEM((CA,), jnp.int32),
        pltpu.VMEM((CA,), jnp.int32),
        pltpu.VMEM((CA,), jnp.int32),
        pltpu.VMEM((BUFSZ,), jnp.int32),
        pltpu.VMEM((Q,), jnp.int32),
        pltpu.VMEM((Q,), jnp.int32),
        pltpu.VMEM((Q,), jnp.int32),
        pltpu.VMEM((Q,), jnp.int32),
        pltpu.VMEM((Q, W80), jnp.float32),
        pltpu.VMEM((Q, W80), jnp.float32),
        pltpu.VMEM((Q,), jnp.int32),
        pltpu.VMEM_SHARED((ACCROWS, W80), jnp.float32),
        pltpu.SemaphoreType.DMA,
        pltpu.SemaphoreType.DMA,
    ],
)
def _sc_edges(map_h, src_h, dst_h, typ_h, aug_h, eid_h, arange_h, out_h,
              map_v, src_v, dst_v, typ_v, pack_b,
              idx_q0, aug_q0, idx_q1, aug_q1, stg0, stg1,
              gidx_v, acc, sem0, sem1):
    cidx = lax.axis_index("c")
    sidx = lax.axis_index("s")
    wid = sidx * NC + cidx

    zero16 = jnp.zeros((16,), jnp.float32)

    def zrow(i, carry):
        for cw in range(W80 // 16):
            stg0[i, pl.ds(cw * 16, 16)] = zero16
        return carry

    lax.fori_loop(0, Q, zrow, 0)
    zbase = sidx * (ACCROWS // NS)
    for k in range(2):
        pltpu.sync_copy(stg0, acc.at[pl.ds(zbase + k * Q, Q)])
    pltpu.sync_copy(stg0.at[pl.ds(0, 16)], acc.at[pl.ds(zbase + 2 * Q, 16)])

    pltpu.sync_copy(map_h, map_v)
    pltpu.sync_copy(eid_h, src_v.at[pl.ds(0, B)])
    pltpu.sync_copy(arange_h, dst_v.at[pl.ds(0, B)])

    def mbody(r_, carry):
        e = src_v[pl.ds(r_ * 16, 16)]
        v = dst_v[pl.ds(r_ * 16, 16)]
        plsc.store_scatter(map_v, [e], v)
        return carry

    lax.fori_loop(0, B // 16, mbody, 0)
    plsc.subcore_barrier()

    idx_sets = ((idx_q0, aug_q0, stg0, sem0), (idx_q1, aug_q1, stg1, sem1))

    def issue(q, k):
        iq, aq, stg, sem = idx_sets[k]
        for r_ in range(Q // 16):
            p = pack_b[pl.ds(q * Q + r_ * 16, 16)]
            iq[pl.ds(r_ * 16, 16)] = lax.shift_right_logical(p, 11)
            aq[pl.ds(r_ * 16, 16)] = lax.bitwise_and(p, 2047)
        pltpu.async_copy(aug_h.at[aq], stg, sem)

    def drain_scatter(q, k, nact):
        iq, aq, stg, sem = idx_sets[k]
        pltpu.make_async_copy(aug_h.at[aq], stg, sem).wait()
        pltpu.sync_copy(stg, acc.at[iq], add=True)

        @pl.when(q + 2 < nact)
        def _():
            issue(q + 2, k)

    def run_chunk(cb, csz):
        pltpu.sync_copy(src_h.at[pl.ds(cb, csz)], src_v.at[pl.ds(0, csz)])
        pltpu.sync_copy(dst_h.at[pl.ds(cb, csz)], dst_v.at[pl.ds(0, csz)])
        pltpu.sync_copy(typ_h.at[pl.ds(cb, csz)], typ_v.at[pl.ds(0, csz)])

        def vbody(v, w):
            off = v * 16
            s = src_v[pl.ds(off, 16)]
            d = dst_v[pl.ds(off, 16)]
            t = typ_v[pl.ds(off, 16)]
            ss = plsc.load_gather(map_v, [s])
            sd = plsc.load_gather(map_v, [d])
            selfm = s == d
            ms = ss >= 0
            md = jnp.logical_and(sd >= 0, jnp.logical_not(selfm))
            pack_s = ss * 2048 + jnp.where(selfm, t + R, t)
            pack_d = sd * 2048 + t
            plsc.store_compressed(pack_b.at[pl.ds(w, 16)], pack_s, mask=ms)
            w = w + _popcount(ms)
            plsc.store_compressed(pack_b.at[pl.ds(w, 16)], pack_d, mask=md)
            w = w + _popcount(md)
            return w

        w = lax.fori_loop(0, csz // 16, vbody, jnp.int32(0))

        trash = jnp.full((16,), PACK_TRASH, jnp.int32)
        for k in range(Q // 16):
            pack_b[pl.ds(w + k * 16, 16)] = trash
        nact = lax.div(w + (Q - 1), jnp.int32(Q))

        @pl.when(nact > 0)
        def _():
            issue(jnp.int32(0), 0)

        @pl.when(nact > 1)
        def _():
            issue(jnp.int32(1), 1)

        def pair_body(j2, carry):
            q0 = j2 * 2

            @pl.when(q0 < nact)
            def _():
                drain_scatter(q0, 0, nact)

            @pl.when(q0 + 1 < nact)
            def _():
                drain_scatter(q0 + 1, 1, nact)

            return carry

        lax.fori_loop(0, lax.div(nact + 1, jnp.int32(2)), pair_body, 0)

    @pl.when(wid < NW - 1)
    def _():
        for ch in range(NCH):
            run_chunk(wid * WEA + ch * CA, CA)

    @pl.when(wid == NW - 1)
    def _():
        for ch in range(NCH):
            run_chunk((NW - 1) * WEA + ch * CB, CB)

    plsc.subcore_barrier()
    rb = sidx * (B // NS)
    for h in range(2):
        hb = rb + h * Q
        pltpu.sync_copy(eid_h.at[pl.ds(hb, Q)], idx_q0)

        def gb(r_, carry):
            e = idx_q0[pl.ds(r_ * 16, 16)]
            gidx_v[pl.ds(r_ * 16, 16)] = plsc.load_gather(map_v, [e])
            return carry

        lax.fori_loop(0, Q // 16, gb, 0)

        pltpu.async_copy(acc.at[gidx_v], stg0, sem0).wait()
        pltpu.sync_copy(stg0, out_h.at[cidx, pl.ds(hb, Q)])


def _tc_body(comb_ref, remb_ref, wi1_ref, bi1_ref, wi2_ref, bi2_ref,
             wa1_ref, ba1_ref, wa2_ref, ba2_ref, st_ref, out_ref):
    comb = comb_ref[0] + comb_ref[1]
    x = comb[:, :D]
    cnt = comb[:, D:D + 1]
    nbr = comb[:, D + 1:D + 2]
    remb = remb_ref[...]
    rc = jnp.mean(remb, axis=0, keepdims=True)
    xa = x / jnp.maximum(cnt, 1.0)
    wi1 = wi1_ref[...]
    pre1 = (jnp.dot(xa, wi1[:D], preferred_element_type=jnp.float32)
            + jnp.dot(rc, wi1[D:], preferred_element_type=jnp.float32)
            + bi1_ref[...])
    h1 = (jnp.dot(jax.nn.relu(pre1), wi2_ref[...],
                  preferred_element_type=jnp.float32) + bi2_ref[...])
    wa1 = wa1_ref[...]
    pre2 = (jnp.dot(xa, wa1[:D] + wa1[D:],
                    preferred_element_type=jnp.float32) + ba1_ref[...])
    h2 = (jnp.dot(jax.nn.relu(pre2), wa2_ref[...],
                  preferred_element_type=jnp.float32) + ba2_ref[...])
    ctx = jnp.where(nbr > 0.0, h2, h1)
    alpha = jnp.clip(st_ref[0, 0], 0.0, 0.3)
    f = (1.0 - alpha) * xa + alpha * ctx
    out_ref[...] = jnp.where(cnt > 0.0, f, jnp.broadcast_to(rc, f.shape))


_tc_mlp = pl.pallas_call(
    _tc_body,
    out_shape=jax.ShapeDtypeStruct((B, D), jnp.float32),
)


def kernel(entity_ids, edge_index, edge_type, relation_embeddings,
           Wi1, bi1, Wi2, bi2, Wa1, ba1, Wa2, ba2, strength):
    eids = entity_ids.astype(jnp.int32)
    src = edge_index[0].astype(jnp.int32)
    dst = edge_index[1].astype(jnp.int32)
    typ = edge_type.astype(jnp.int32)
    remb = relation_embeddings.astype(jnp.float32)
    aug = jnp.zeros((AUGROWS, W80), jnp.float32)
    aug = aug.at[:R, :D].set(remb)
    aug = aug.at[:R, D].set(1.0)
    aug = aug.at[:R, D + 1].set(1.0)
    aug = aug.at[R:2 * R, :D].set(remb)
    aug = aug.at[R:2 * R, D].set(1.0)
    map_base = jnp.full((NPAD,), -1, jnp.int32)
    slot_vals = jnp.arange(B, dtype=jnp.int32)

    partials = _sc_edges(map_base, src, dst, typ, aug, eids, slot_vals)
    return _tc_mlp(partials, remb, Wi1,
                   bi1.reshape(1, D), Wi2, bi2.reshape(1, D),
                   Wa1, ba1.reshape(1, D), Wa2, ba2.reshape(1, D),
                   strength.reshape(1, 1).astype(jnp.float32))
